# Initial kernel scaffold; baseline (speedup 1.0000x reference)
import jax
import jax.numpy as jnp
from jax.experimental import pallas as pl


def kernel(x, Wq, bq, Wk, bk, Wv, bv, Wo, bo):
    raise NotImplementedError("write your pallas kernel here")


# speedup vs baseline: 10.5856x; 10.5856x over previous
import math

import jax
import jax.numpy as jnp
from jax import lax
from jax.experimental import pallas as pl

_TOPK = 32
_MIN32_PY = -(2 ** 31)


def _proj_body(x_ref, w_ref, b_ref, o_ref):
    o_ref[...] = (
        jnp.dot(x_ref[...], w_ref[...], preferred_element_type=jnp.float32)
        + b_ref[...]
    )


def _attn_body(q_ref, kt_ref, v_ref, wo_ref, bo_ref, o_ref, *, scale, topk):
    q = q_ref[0]
    kt = kt_ref[0]
    s = jnp.dot(q, kt, preferred_element_type=jnp.float32) * scale
    t_rows = s.shape[0]

    m = jnp.max(s, axis=1, keepdims=True)
    min32 = jnp.int32(_MIN32_PY)

    bi = lax.bitcast_convert_type(s, jnp.int32)
    ks = jnp.bitwise_xor(
        bi, jnp.bitwise_or(lax.shift_right_arithmetic(bi, 31), min32)
    )

    prefix = jnp.zeros((t_rows, 1), jnp.int32)
    kneed = jnp.full((t_rows, 1), topk, jnp.int32)
    for j in range(31, -1, -1):
        bit = min32 if j == 31 else jnp.int32(1 << j)
        hi = lax.shift_right_logical(ks, j)
        target = jnp.bitwise_or(lax.shift_right_logical(prefix, j),
                                jnp.int32(1))
        pred = hi == target
        c = jnp.sum(pred.astype(jnp.int32), axis=1, keepdims=True)
        take = c >= kneed
        prefix = jnp.where(take, jnp.bitwise_or(prefix, bit), prefix)
        kneed = jnp.where(take, kneed, kneed - c)

    keep = jnp.bitwise_xor(ks, min32) >= jnp.bitwise_xor(prefix, min32)
    p = jnp.where(keep, jnp.exp(s - m), 0.0)
    r = jnp.sum(p, axis=1, keepdims=True)

    ao = jnp.dot(p, v_ref[0], preferred_element_type=jnp.float32) / r
    o_ref[0] = (
        jnp.dot(ao, wo_ref[...], preferred_element_type=jnp.float32)
        + bo_ref[...]
    )


def kernel(x, Wq, bq, Wk, bk, Wv, bv, Wo, bo):
    B, S, D = x.shape
    H = Wq.shape[1]
    scale = 1.0 / math.sqrt(H)
    TP = 512
    T = 256

    Wqkv = jnp.concatenate([Wq, Wk, Wv], axis=1)
    bqkv = jnp.concatenate([bq, bk, bv]).reshape(1, -1)
    P = 2 * H + D
    xf = x.reshape(B * S, D)

    qkv = pl.pallas_call(
        _proj_body,
        grid=(B * S // TP,),
        in_specs=[
            pl.BlockSpec((TP, D), lambda i: (i, 0)),
            pl.BlockSpec((D, P), lambda i: (0, 0)),
            pl.BlockSpec((1, P), lambda i: (0, 0)),
        ],
        out_specs=pl.BlockSpec((TP, P), lambda i: (i, 0)),
        out_shape=jax.ShapeDtypeStruct((B * S, P), jnp.float32),
    )(xf, Wqkv, bqkv)

    Q = qkv[:, :H].reshape(B, S, H)
    KT = jnp.swapaxes(qkv[:, H:2 * H].reshape(B, S, H), 1, 2)
    V = qkv[:, 2 * H:].reshape(B, S, D)

    import functools
    body = functools.partial(_attn_body, scale=scale, topk=_TOPK)
    out = pl.pallas_call(
        body,
        grid=(B, S // T),
        in_specs=[
            pl.BlockSpec((1, T, H), lambda b, i: (b, i, 0)),
            pl.BlockSpec((1, H, S), lambda b, i: (b, 0, 0)),
            pl.BlockSpec((1, S, D), lambda b, i: (b, 0, 0)),
            pl.BlockSpec((D, D), lambda b, i: (0, 0)),
            pl.BlockSpec((1, D), lambda b, i: (0, 0)),
        ],
        out_specs=pl.BlockSpec((1, T, D), lambda b, i: (b, i, 0)),
        out_shape=jax.ShapeDtypeStruct((B, S, D), jnp.float32),
    )(Q, KT, V, Wo, bo.reshape(1, D))
    return out


# speedup vs baseline: 14.1952x; 1.3410x over previous
import functools
import math

import jax
import jax.numpy as jnp
from jax import lax
from jax.experimental import pallas as pl

_TOPK = 32
_SELBITS = 22


def _fold_body(wv_ref, wo_ref, bv_ref, bo_ref, w2_ref, b2_ref):
    w2_ref[...] = jnp.dot(wv_ref[...], wo_ref[...],
                          preferred_element_type=jnp.float32)
    b2_ref[...] = jnp.dot(bv_ref[...], wo_ref[...],
                          preferred_element_type=jnp.float32) + bo_ref[...]


def _proj_body(x_ref, w_ref, b_ref, qk_ref, u_ref, *, h2):
    o = (jnp.dot(x_ref[...], w_ref[...], preferred_element_type=jnp.float32)
         + b_ref[...])
    qk_ref[...] = o[:, :h2]
    u_ref[...] = o[:, h2:].astype(jnp.bfloat16)


def _attn_body(q_ref, kt_ref, u_ref, b2_ref, o_ref, *, scale, topk, selbits):
    q = q_ref[0]
    kt = kt_ref[0]
    s = jnp.dot(q, kt, preferred_element_type=jnp.float32) * scale
    t_rows = s.shape[0]

    m = jnp.max(s, axis=1, keepdims=True)
    lo = jnp.min(s, axis=1, keepdims=True)
    kscale = ((1 << selbits) - 2) / jnp.maximum(m - lo, jnp.float32(1e-30))
    v = ((s - lo) * kscale).astype(jnp.int32)

    prefix = jnp.zeros((t_rows, 1), jnp.int32)
    kneed = jnp.full((t_rows, 1), topk, jnp.int32)
    for j in range(selbits - 1, -1, -1):
        hi = lax.shift_right_logical(v, j)
        target = jnp.bitwise_or(lax.shift_right_logical(prefix, j),
                                jnp.int32(1))
        pred = hi == target
        c = jnp.sum(pred.astype(jnp.int32), axis=1, keepdims=True)
        take = c >= kneed
        prefix = jnp.where(take, jnp.bitwise_or(prefix, jnp.int32(1 << j)),
                           prefix)
        kneed = jnp.where(take, kneed, kneed - c)

    keep = v >= prefix
    p = jnp.where(keep, jnp.exp(s - m), 0.0)
    r = jnp.sum(p, axis=1, keepdims=True)

    ao = jnp.dot(p.astype(jnp.bfloat16), u_ref[0],
                 preferred_element_type=jnp.float32)
    o_ref[0] = ao / r + b2_ref[...]


def kernel(x, Wq, bq, Wk, bk, Wv, bv, Wo, bo):
    B, S, D = x.shape
    H = Wq.shape[1]
    scale = 1.0 / math.sqrt(H)
    TP = 512
    T = 256

    W2, b2 = pl.pallas_call(
        _fold_body,
        out_shape=(jax.ShapeDtypeStruct((D, D), jnp.float32),
                   jax.ShapeDtypeStruct((1, D), jnp.float32)),
    )(Wv, Wo, bv.reshape(1, D), bo.reshape(1, D))

    Wcat = jnp.concatenate([Wq, Wk, W2], axis=1)
    bcat = jnp.concatenate(
        [bq, bk, jnp.zeros((D,), jnp.float32)]).reshape(1, -1)
    P = 2 * H + D
    xf = x.reshape(B * S, D)

    qk, U = pl.pallas_call(
        functools.partial(_proj_body, h2=2 * H),
        grid=(B * S // TP,),
        in_specs=[
            pl.BlockSpec((TP, D), lambda i: (i, 0)),
            pl.BlockSpec((D, P), lambda i: (0, 0)),
            pl.BlockSpec((1, P), lambda i: (0, 0)),
        ],
        out_specs=(pl.BlockSpec((TP, 2 * H), lambda i: (i, 0)),
                   pl.BlockSpec((TP, D), lambda i: (i, 0))),
        out_shape=(jax.ShapeDtypeStruct((B * S, 2 * H), jnp.float32),
                   jax.ShapeDtypeStruct((B * S, D), jnp.bfloat16)),
    )(xf, Wcat, bcat)

    Q = qk[:, :H].reshape(B, S, H)
    KT = jnp.swapaxes(qk[:, H:].reshape(B, S, H), 1, 2)
    Ub = U.reshape(B, S, D)

    body = functools.partial(_attn_body, scale=scale, topk=_TOPK,
                             selbits=_SELBITS)
    out = pl.pallas_call(
        body,
        grid=(B, S // T),
        in_specs=[
            pl.BlockSpec((1, T, H), lambda b, i: (b, i, 0)),
            pl.BlockSpec((1, H, S), lambda b, i: (b, 0, 0)),
            pl.BlockSpec((1, S, D), lambda b, i: (b, 0, 0)),
            pl.BlockSpec((1, D), lambda b, i: (0, 0)),
        ],
        out_specs=pl.BlockSpec((1, T, D), lambda b, i: (b, i, 0)),
        out_shape=jax.ShapeDtypeStruct((B, S, D), jnp.float32),
    )(Q, KT, Ub, b2)
    return out


# speedup vs baseline: 16.7129x; 1.1774x over previous
import functools
import math

import jax
import jax.numpy as jnp
from jax import lax
from jax.experimental import pallas as pl

_TOPK = 32
_SELBITS = 24


def _fold_body(wv_ref, wo_ref, bv_ref, bo_ref, w2_ref, b2_ref):
    w2_ref[...] = jnp.dot(wv_ref[...], wo_ref[...],
                          preferred_element_type=jnp.float32)
    b2_ref[...] = jnp.dot(bv_ref[...], wo_ref[...],
                          preferred_element_type=jnp.float32) + bo_ref[...]


def _proj_body(x_ref, w_ref, b_ref, qk_ref, u_ref, *, h2):
    o = (jnp.dot(x_ref[...], w_ref[...], preferred_element_type=jnp.float32)
         + b_ref[...])
    qk_ref[...] = o[:, :h2]
    u_ref[...] = o[:, h2:].astype(jnp.bfloat16)


def _attn_body(q_ref, kt_ref, u_ref, b2_ref, o_ref, *, scale, topk, selbits):
    q = q_ref[0]
    kt = kt_ref[0]
    s = jnp.dot(q, kt, preferred_element_type=jnp.float32) * scale

    m = jnp.max(s, axis=1, keepdims=True)
    lo = jnp.min(s, axis=1, keepdims=True)

    lo_b = lo
    hi_b = m
    for _ in range(selbits):
        mid = 0.5 * (lo_b + hi_b)
        c = jnp.sum((s >= mid).astype(jnp.int32), axis=1, keepdims=True)
        ge = c >= topk
        lo_b = jnp.where(ge, mid, lo_b)
        hi_b = jnp.where(ge, hi_b, mid)

    keep = s >= lo_b
    p = jnp.where(keep, jnp.exp(s - m), 0.0)
    r = jnp.sum(p, axis=1, keepdims=True)

    ao = jnp.dot(p.astype(jnp.bfloat16), u_ref[0],
                 preferred_element_type=jnp.float32)
    o_ref[0] = ao / r + b2_ref[...]


def kernel(x, Wq, bq, Wk, bk, Wv, bv, Wo, bo):
    B, S, D = x.shape
    H = Wq.shape[1]
    scale = 1.0 / math.sqrt(H)
    TP = 512
    T = 512

    W2, b2 = pl.pallas_call(
        _fold_body,
        out_shape=(jax.ShapeDtypeStruct((D, D), jnp.float32),
                   jax.ShapeDtypeStruct((1, D), jnp.float32)),
    )(Wv, Wo, bv.reshape(1, D), bo.reshape(1, D))

    Wcat = jnp.concatenate([Wq, Wk, W2], axis=1)
    bcat = jnp.concatenate(
        [bq, bk, jnp.zeros((D,), jnp.float32)]).reshape(1, -1)
    P = 2 * H + D
    xf = x.reshape(B * S, D)

    qk, U = pl.pallas_call(
        functools.partial(_proj_body, h2=2 * H),
        grid=(B * S // TP,),
        in_specs=[
            pl.BlockSpec((TP, D), lambda i: (i, 0)),
            pl.BlockSpec((D, P), lambda i: (0, 0)),
            pl.BlockSpec((1, P), lambda i: (0, 0)),
        ],
        out_specs=(pl.BlockSpec((TP, 2 * H), lambda i: (i, 0)),
                   pl.BlockSpec((TP, D), lambda i: (i, 0))),
        out_shape=(jax.ShapeDtypeStruct((B * S, 2 * H), jnp.float32),
                   jax.ShapeDtypeStruct((B * S, D), jnp.bfloat16)),
    )(xf, Wcat, bcat)

    Q = qk[:, :H].reshape(B, S, H)
    KT = jnp.swapaxes(qk[:, H:].reshape(B, S, H), 1, 2)
    Ub = U.reshape(B, S, D)

    body = functools.partial(_attn_body, scale=scale, topk=_TOPK,
                             selbits=_SELBITS)
    out = pl.pallas_call(
        body,
        grid=(B, S // T),
        in_specs=[
            pl.BlockSpec((1, T, H), lambda b, i: (b, i, 0)),
            pl.BlockSpec((1, H, S), lambda b, i: (b, 0, 0)),
            pl.BlockSpec((1, S, D), lambda b, i: (b, 0, 0)),
            pl.BlockSpec((1, D), lambda b, i: (0, 0)),
        ],
        out_specs=pl.BlockSpec((1, T, D), lambda b, i: (b, i, 0)),
        out_shape=jax.ShapeDtypeStruct((B, S, D), jnp.float32),
    )(Q, KT, Ub, b2)
    return out


# speedup vs baseline: 16.8688x; 1.0093x over previous
import functools
import math

import jax
import jax.numpy as jnp
from jax import lax
from jax.experimental import pallas as pl

_TOPK = 32
_SELBITS = 24


def _fold_body(wv_ref, wo_ref, bv_ref, bo_ref, w2_ref, b2_ref):
    w2_ref[...] = jnp.dot(wv_ref[...], wo_ref[...],
                          preferred_element_type=jnp.float32)
    b2_ref[...] = jnp.dot(bv_ref[...], wo_ref[...],
                          preferred_element_type=jnp.float32) + bo_ref[...]


def _proj_body(x_ref, w_ref, b_ref, q_ref, k_ref, u_ref, *, h):
    o = (jnp.dot(x_ref[...], w_ref[...], preferred_element_type=jnp.float32)
         + b_ref[...])
    q_ref[0] = o[:, :h]
    k_ref[0] = o[:, h:2 * h]
    u_ref[0] = o[:, 2 * h:].astype(jnp.bfloat16)


def _attn_body(q_ref, k_ref, u_ref, b2_ref, o_ref, *, scale, topk, selbits):
    q = q_ref[0]
    k = k_ref[0]
    s = lax.dot_general(q, k, (((1,), (1,)), ((), ())),
                        preferred_element_type=jnp.float32) * scale

    m = jnp.max(s, axis=1, keepdims=True)
    lo = jnp.min(s, axis=1, keepdims=True)

    lo_b = lo
    hi_b = m
    for _ in range(selbits):
        mid = 0.5 * (lo_b + hi_b)
        c = jnp.sum((s >= mid).astype(jnp.int32), axis=1, keepdims=True)
        ge = c >= topk
        lo_b = jnp.where(ge, mid, lo_b)
        hi_b = jnp.where(ge, hi_b, mid)

    keep = s >= lo_b
    p = jnp.where(keep, jnp.exp(s - m), 0.0)
    r = jnp.sum(p, axis=1, keepdims=True)

    ao = jnp.dot(p.astype(jnp.bfloat16), u_ref[0],
                 preferred_element_type=jnp.float32)
    o_ref[0] = ao / r + b2_ref[...]


def kernel(x, Wq, bq, Wk, bk, Wv, bv, Wo, bo):
    B, S, D = x.shape
    H = Wq.shape[1]
    scale = 1.0 / math.sqrt(H)
    TP = 512
    T = 512

    W2, b2 = pl.pallas_call(
        _fold_body,
        out_shape=(jax.ShapeDtypeStruct((D, D), jnp.float32),
                   jax.ShapeDtypeStruct((1, D), jnp.float32)),
    )(Wv, Wo, bv.reshape(1, D), bo.reshape(1, D))

    Wcat = jnp.concatenate([Wq, Wk, W2], axis=1)
    bcat = jnp.concatenate(
        [bq, bk, jnp.zeros((D,), jnp.float32)]).reshape(1, -1)
    P = 2 * H + D
    xf = x.reshape(B * S, D)

    npb = S // TP
    Q, K, U = pl.pallas_call(
        functools.partial(_proj_body, h=H),
        grid=(B * S // TP,),
        in_specs=[
            pl.BlockSpec((TP, D), lambda i: (i, 0)),
            pl.BlockSpec((D, P), lambda i: (0, 0)),
            pl.BlockSpec((1, P), lambda i: (0, 0)),
        ],
        out_specs=(
            pl.BlockSpec((1, TP, H), lambda i: (i // npb, i % npb, 0)),
            pl.BlockSpec((1, TP, H), lambda i: (i // npb, i % npb, 0)),
            pl.BlockSpec((1, TP, D), lambda i: (i // npb, i % npb, 0)),
        ),
        out_shape=(jax.ShapeDtypeStruct((B, S, H), jnp.float32),
                   jax.ShapeDtypeStruct((B, S, H), jnp.float32),
                   jax.ShapeDtypeStruct((B, S, D), jnp.bfloat16)),
    )(xf, Wcat, bcat)

    body = functools.partial(_attn_body, scale=scale, topk=_TOPK,
                             selbits=_SELBITS)
    out = pl.pallas_call(
        body,
        grid=(B, S // T),
        in_specs=[
            pl.BlockSpec((1, T, H), lambda b, i: (b, i, 0)),
            pl.BlockSpec((1, S, H), lambda b, i: (b, 0, 0)),
            pl.BlockSpec((1, S, D), lambda b, i: (b, 0, 0)),
            pl.BlockSpec((1, D), lambda b, i: (0, 0)),
        ],
        out_specs=pl.BlockSpec((1, T, D), lambda b, i: (b, i, 0)),
        out_shape=jax.ShapeDtypeStruct((B, S, D), jnp.float32),
    )(Q, K, U, b2)
    return out
